# baseline (speedup 1.0000x reference)
import functools

import jax
import jax.numpy as jnp
from jax.experimental import pallas as pl
from jax.experimental.pallas import tpu as pltpu

_EPS = 1e-5


def _stage_taps(x4, x9_ref, h, w):
    s, _, _, c = x4.shape
    xp = jnp.pad(x4, ((0, 0), (1, 1), (1, 1), (0, 0)))
    k = 0
    for dy in range(3):
        for dx in range(3):
            x9_ref[:, k * c:(k + 1) * c] = (
                xp[:, dy:dy + h, dx:dx + w, :].reshape(s * h * w, c))
            k += 1


def _acc_stats(y, ps, pq):
    m, c = y.shape
    y8 = y.reshape(m // 8, 8, c)
    return ps + jnp.sum(y8, axis=0), pq + jnp.sum(y8 * y8, axis=0)


def _convs_kernel(x_ref, w_ref, bn_ref, y2_ref, ps_ref, pq_ref,
                  ybuf, x9_ref, stat, aff,
                  *, h, w, blk, sub, nb, n, cin, c1, c2):
    hw = h * w
    p = pl.program_id(0)
    i = pl.program_id(1)

    @pl.when((p == 0) & (i == 0))
    def _():
        stat[...] = jnp.zeros(stat.shape, stat.dtype)

    @pl.when(p == 0)
    def _():
        ps = jnp.zeros((8, c1), jnp.float32)
        pq = jnp.zeros((8, c1), jnp.float32)
        for j in range(0, blk, sub):
            xs = x_ref[j:j + sub]
            xt = jnp.swapaxes(xs, 1, 2).astype(jnp.bfloat16)
            _stage_taps(xt.reshape(sub, h, w, cin), x9_ref, h, w)
            y = jnp.dot(x9_ref[:, :9 * cin], w_ref[:9 * cin],
                        preferred_element_type=jnp.float32)
            ps, pq = _acc_stats(y, ps, pq)
            ybuf[pl.ds(i * blk + j, sub)] = (
                y.reshape(sub, hw, c1).astype(jnp.bfloat16))
        stat[0] += ps
        stat[1] += pq

    @pl.when((p == 1) & (i == 0))
    def _():
        m_total = float(n * hw)
        s = jnp.sum(stat[0], axis=0, keepdims=True)
        q = jnp.sum(stat[1], axis=0, keepdims=True)
        mean = s / m_total
        var = q / m_total - mean * mean
        sc = bn_ref[0:1] * jax.lax.rsqrt(var + _EPS)
        aff[0:1] = sc
        aff[1:2] = bn_ref[1:2] - mean * sc

    @pl.when(p == 1)
    def _():
        ps = jnp.zeros((8, c2), jnp.float32)
        pq = jnp.zeros((8, c2), jnp.float32)
        for j in range(0, blk, sub):
            v = ybuf[pl.ds(i * blk + j, sub)].astype(jnp.float32)
            hr = jnp.maximum(v * aff[0:1] + aff[1:2], 0.0).astype(jnp.bfloat16)
            _stage_taps(hr.reshape(sub, h, w, c1), x9_ref, h, w)
            y = jnp.dot(x9_ref[...], w_ref[9 * cin:],
                        preferred_element_type=jnp.float32)
            ps, pq = _acc_stats(y, ps, pq)
            y2_ref[j:j + sub] = y.reshape(sub, hw, c2).astype(jnp.bfloat16)
        ps_ref[0] = jnp.sum(ps, axis=0, keepdims=True)
        pq_ref[0] = jnp.sum(pq, axis=0, keepdims=True)


def _convs(x3, wcat, bn, h, w, blk, sub, cin, c1, c2):
    n = x3.shape[0]
    hw = h * w
    nb = n // blk

    def x_idx(p, i):
        return (jnp.where(p == 0, i, 0), 0, 0)

    def y_idx(p, i):
        return (jnp.where(p == 1, i, 0), 0, 0)

    return pl.pallas_call(
        functools.partial(_convs_kernel, h=h, w=w, blk=blk, sub=sub, nb=nb,
                          n=n, cin=cin, c1=c1, c2=c2),
        grid=(2, nb),
        in_specs=[
            pl.BlockSpec((blk, cin, hw), x_idx),
            pl.BlockSpec((9 * cin + 9 * c1, c2), lambda p, i: (0, 0)),
            pl.BlockSpec((2, c1), lambda p, i: (0, 0)),
        ],
        out_specs=(
            pl.BlockSpec((blk, hw, c2), y_idx),
            pl.BlockSpec((1, 1, c2), y_idx),
            pl.BlockSpec((1, 1, c2), y_idx),
        ),
        out_shape=(
            jax.ShapeDtypeStruct((n, hw, c2), jnp.bfloat16),
            jax.ShapeDtypeStruct((n // blk, 1, c2), jnp.float32),
            jax.ShapeDtypeStruct((n // blk, 1, c2), jnp.float32),
        ),
        scratch_shapes=[
            pltpu.VMEM((n, hw, c1), jnp.bfloat16),
            pltpu.VMEM((sub * hw, 9 * c1), jnp.bfloat16),
            pltpu.VMEM((2, 8, c1), jnp.float32),
            pltpu.VMEM((2, c1), jnp.float32),
        ],
        compiler_params=pltpu.CompilerParams(
            dimension_semantics=("arbitrary", "arbitrary"),
            vmem_limit_bytes=60000 * 1024),
    )(x3, wcat, bn)


def _epi_kernel(y2_ref, s_ref, t_ref, down_ref, pool_ref, rp_ref, *, h, w):
    b, hw, c = y2_ref.shape
    d = jnp.maximum(
        y2_ref[...].astype(jnp.float32) * s_ref[...] + t_ref[...], 0.0)
    down_ref[...] = jnp.swapaxes(d, 1, 2)
    half = d.reshape(b * (h // 2), 2 * w, c)
    rp = jnp.maximum(half[:, :w, :], half[:, w:, :])
    rp_ref[...] = rp.reshape(b * (hw // 2), c)
    npool = b * (hw // 4)
    pr = jnp.maximum(rp_ref[pl.ds(0, npool, 2), :],
                     rp_ref[pl.ds(1, npool, 2), :])
    pool_ref[...] = jnp.swapaxes(pr.reshape(b, hw // 4, c), 1, 2)


def _epi(y2, sc, sh, h, w, blk):
    n, hw, c = y2.shape
    nb = n // blk
    vec = pl.BlockSpec((1, c), lambda i: (0, 0))
    return pl.pallas_call(
        functools.partial(_epi_kernel, h=h, w=w),
        grid=(nb,),
        in_specs=[pl.BlockSpec((blk, hw, c), lambda i: (i, 0, 0)), vec, vec],
        out_specs=(
            pl.BlockSpec((blk, c, hw), lambda i: (i, 0, 0)),
            pl.BlockSpec((blk, c, hw // 4), lambda i: (i, 0, 0)),
        ),
        out_shape=(
            jax.ShapeDtypeStruct((n, c, hw), jnp.float32),
            jax.ShapeDtypeStruct((n, c, hw // 4), jnp.float32),
        ),
        scratch_shapes=[pltpu.VMEM((blk * hw // 2, c), jnp.float32)],
        compiler_params=pltpu.CompilerParams(
            dimension_semantics=("arbitrary",)),
    )(y2, sc, sh)


def kernel(x, w1, b1, g1, beta1, w2, b2, g2, beta2):
    n, cin, h, w = x.shape
    c1 = w1.shape[-1]
    c2 = w2.shape[-1]
    hw = h * w
    x3 = x.reshape(n, cin, hw)
    w1f = w1.reshape(9 * cin, c1).astype(jnp.bfloat16)
    w2f = w2.reshape(9 * c1, c2).astype(jnp.bfloat16)
    wcat = jnp.concatenate([w1f, w2f], axis=0)
    bn1 = jnp.concatenate([g1, beta1], axis=0)
    m = float(n * hw)

    blk = min(8, n)
    sub = min(2, blk)
    y2, ps2, pq2 = _convs(x3, wcat, bn1, h, w, blk, sub, cin, c1, c2)
    s2 = jnp.sum(ps2, axis=0)
    q2 = jnp.sum(pq2, axis=0)
    mean = s2 / m
    var = q2 / m - mean * mean
    sc2 = g2 * jax.lax.rsqrt(var + _EPS)
    sh2 = beta2 - mean * sc2
    down, pool = _epi(y2, sc2, sh2, h, w, min(8, n))
    return (down.reshape(n, c2, h, w), pool.reshape(n, c2, h // 2, w // 2))


# speedup vs baseline: 1.0025x; 1.0025x over previous
import functools

import jax
import jax.numpy as jnp
from jax.experimental import pallas as pl
from jax.experimental.pallas import tpu as pltpu

_EPS = 1e-5


def _stage_taps(x4, x9_ref, h, w):
    s, _, _, c = x4.shape
    xp = jnp.pad(x4, ((0, 0), (1, 1), (1, 1), (0, 0)))
    for k, (dy, dx) in enumerate((dy, dx) for dy in range(3) for dx in range(3)):
        x9_ref[:, k * c:(k + 1) * c] = (
            xp[:, dy:dy + h, dx:dx + w, :].reshape(s * h * w, c))


def _acc_stats(y, ps, pq):
    m, c = y.shape
    y8 = y.reshape(m // 8, 8, c)
    return ps + jnp.sum(y8, axis=0), pq + jnp.sum(y8 * y8, axis=0)


def _conv1_kernel(x_ref, w_ref, y_ref, ps_ref, pq_ref, x9_ref, *, h, w, sub):
    b, c, hw = x_ref.shape
    cout = w_ref.shape[-1]
    ps = jnp.zeros((8, cout), jnp.float32)
    pq = jnp.zeros((8, cout), jnp.float32)
    for j in range(0, b, sub):
        xs = x_ref[j:j + sub]
        xt = jnp.swapaxes(xs, 1, 2).astype(jnp.bfloat16)
        _stage_taps(xt.reshape(sub, h, w, c), x9_ref, h, w)
        y = jnp.dot(x9_ref[...], w_ref[...], preferred_element_type=jnp.float32)
        ps, pq = _acc_stats(y, ps, pq)
        y_ref[j:j + sub] = y.reshape(sub, hw, cout).astype(jnp.bfloat16)
    ps_ref[0] = jnp.sum(ps, axis=0, keepdims=True)
    pq_ref[0] = jnp.sum(pq, axis=0, keepdims=True)


def _conv2_kernel(y1_ref, s_ref, t_ref, w_ref, y_ref, ps_ref, pq_ref,
                  x9_ref, *, h, w, sub):
    b, hw, c = y1_ref.shape
    cout = w_ref.shape[-1]
    ps = jnp.zeros((8, cout), jnp.float32)
    pq = jnp.zeros((8, cout), jnp.float32)
    for j in range(0, b, sub):
        v = y1_ref[j:j + sub].astype(jnp.float32)
        hr = jnp.maximum(v * s_ref[...] + t_ref[...], 0.0).astype(jnp.bfloat16)
        _stage_taps(hr.reshape(sub, h, w, c), x9_ref, h, w)
        y = jnp.dot(x9_ref[...], w_ref[...], preferred_element_type=jnp.float32)
        ps, pq = _acc_stats(y, ps, pq)
        y_ref[j:j + sub] = y.reshape(sub, hw, cout).astype(jnp.bfloat16)
    ps_ref[0] = jnp.sum(ps, axis=0, keepdims=True)
    pq_ref[0] = jnp.sum(pq, axis=0, keepdims=True)


def _epi_kernel(y2_ref, s_ref, t_ref, down_ref, pool_ref, rp_ref, *, h, w):
    b, hw, c = y2_ref.shape
    d = jnp.maximum(
        y2_ref[...].astype(jnp.float32) * s_ref[...] + t_ref[...], 0.0)
    down_ref[...] = jnp.swapaxes(d, 1, 2)
    half = d.reshape(b * (h // 2), 2 * w, c)
    rp = jnp.maximum(half[:, :w, :], half[:, w:, :])
    rp_ref[...] = rp.reshape(b * (hw // 2), c)
    npool = b * (hw // 4)
    pr = jnp.maximum(rp_ref[pl.ds(0, npool, 2), :],
                     rp_ref[pl.ds(1, npool, 2), :])
    pool_ref[...] = jnp.swapaxes(pr.reshape(b, hw // 4, c), 1, 2)


def _conv1(x3, w1f, h, w, blk, sub):
    n, c, hw = x3.shape
    cout = w1f.shape[-1]
    nb = n // blk
    return pl.pallas_call(
        functools.partial(_conv1_kernel, h=h, w=w, sub=sub),
        grid=(nb,),
        in_specs=[
            pl.BlockSpec((blk, c, hw), lambda i: (i, 0, 0)),
            pl.BlockSpec((9 * c, cout), lambda i: (0, 0)),
        ],
        out_specs=(
            pl.BlockSpec((blk, hw, cout), lambda i: (i, 0, 0)),
            pl.BlockSpec((1, 1, cout), lambda i: (i, 0, 0)),
            pl.BlockSpec((1, 1, cout), lambda i: (i, 0, 0)),
        ),
        out_shape=(
            jax.ShapeDtypeStruct((n, hw, cout), jnp.bfloat16),
            jax.ShapeDtypeStruct((nb, 1, cout), jnp.float32),
            jax.ShapeDtypeStruct((nb, 1, cout), jnp.float32),
        ),
        scratch_shapes=[pltpu.VMEM((sub * hw, 9 * c), jnp.bfloat16)],
        compiler_params=pltpu.CompilerParams(
            dimension_semantics=("parallel",)),
    )(x3, w1f)


def _conv2(y1, sc, sh, w2f, h, w, blk, sub):
    n, hw, c = y1.shape
    cout = w2f.shape[-1]
    nb = n // blk
    vec = pl.BlockSpec((1, c), lambda i: (0, 0))
    return pl.pallas_call(
        functools.partial(_conv2_kernel, h=h, w=w, sub=sub),
        grid=(nb,),
        in_specs=[
            pl.BlockSpec((blk, hw, c), lambda i: (i, 0, 0)),
            vec, vec,
            pl.BlockSpec((9 * c, cout), lambda i: (0, 0)),
        ],
        out_specs=(
            pl.BlockSpec((blk, hw, cout), lambda i: (i, 0, 0)),
            pl.BlockSpec((1, 1, cout), lambda i: (i, 0, 0)),
            pl.BlockSpec((1, 1, cout), lambda i: (i, 0, 0)),
        ),
        out_shape=(
            jax.ShapeDtypeStruct((n, hw, cout), jnp.bfloat16),
            jax.ShapeDtypeStruct((nb, 1, cout), jnp.float32),
            jax.ShapeDtypeStruct((nb, 1, cout), jnp.float32),
        ),
        scratch_shapes=[pltpu.VMEM((sub * hw, 9 * c), jnp.bfloat16)],
        compiler_params=pltpu.CompilerParams(
            dimension_semantics=("parallel",)),
    )(y1, sc, sh, w2f)


def _epi(y2, sc, sh, h, w, blk):
    n, hw, c = y2.shape
    nb = n // blk
    vec = pl.BlockSpec((1, c), lambda i: (0, 0))
    return pl.pallas_call(
        functools.partial(_epi_kernel, h=h, w=w),
        grid=(nb,),
        in_specs=[pl.BlockSpec((blk, hw, c), lambda i: (i, 0, 0)), vec, vec],
        out_specs=(
            pl.BlockSpec((blk, c, hw), lambda i: (i, 0, 0)),
            pl.BlockSpec((blk, c, hw // 4), lambda i: (i, 0, 0)),
        ),
        out_shape=(
            jax.ShapeDtypeStruct((n, c, hw), jnp.float32),
            jax.ShapeDtypeStruct((n, c, hw // 4), jnp.float32),
        ),
        scratch_shapes=[pltpu.VMEM((blk * hw // 2, c), jnp.float32)],
        compiler_params=pltpu.CompilerParams(
            dimension_semantics=("parallel",)),
    )(y2, sc, sh)


def _bn_affine(ps, pq, g, beta, m):
    s = jnp.sum(ps, axis=0)
    q = jnp.sum(pq, axis=0)
    mean = s / m
    var = q / m - mean * mean
    sc = g * jax.lax.rsqrt(var + _EPS)
    sh = beta - mean * sc
    return sc, sh


def kernel(x, w1, b1, g1, beta1, w2, b2, g2, beta2):
    n, cin, h, w = x.shape
    c1 = w1.shape[-1]
    c2 = w2.shape[-1]
    hw = h * w
    x3 = x.reshape(n, cin, hw)
    w1f = w1.reshape(9 * cin, c1).astype(jnp.bfloat16)
    w2f = w2.reshape(9 * c1, c2).astype(jnp.bfloat16)
    m = float(n * hw)

    blk = min(16, n)
    sub = min(4, blk)
    y1, ps1, pq1 = _conv1(x3, w1f, h, w, blk, sub)
    sc1, sh1 = _bn_affine(ps1, pq1, g1, beta1, m)
    y2, ps2, pq2 = _conv2(y1, sc1, sh1, w2f, h, w, blk, sub)
    sc2, sh2 = _bn_affine(ps2, pq2, g2, beta2, m)
    down, pool = _epi(y2, sc2, sh2, h, w, min(8, n))
    return (down.reshape(n, c2, h, w), pool.reshape(n, c2, h // 2, w // 2))


# speedup vs baseline: 1.2503x; 1.2472x over previous
import functools

import jax
import jax.numpy as jnp
from jax.experimental import pallas as pl
from jax.experimental.pallas import tpu as pltpu

_EPS = 1e-5


def _taps(x4, h, w):
    s, _, _, c = x4.shape
    xp = jnp.pad(x4, ((0, 0), (1, 1), (1, 1), (0, 0)))
    cols = [xp[:, dy:dy + h, dx:dx + w, :].reshape(s * h * w, c)
            for dy in range(3) for dx in range(3)]
    return jnp.concatenate(cols, axis=-1)


def _acc_stats(y, ps, pq):
    m, c = y.shape
    y8 = y.reshape(m // 8, 8, c)
    return ps + jnp.sum(y8, axis=0), pq + jnp.sum(y8 * y8, axis=0)


def _conv1_kernel(x_ref, w_ref, y_ref, ps_ref, pq_ref, *, h, w, sub):
    b, c, hw = x_ref.shape
    cout = w_ref.shape[-1]
    ps = jnp.zeros((8, cout), jnp.float32)
    pq = jnp.zeros((8, cout), jnp.float32)
    for j in range(0, b, sub):
        xs = x_ref[j:j + sub]
        xt = jnp.swapaxes(xs, 1, 2).astype(jnp.bfloat16)
        x9 = _taps(xt.reshape(sub, h, w, c), h, w)
        y = jnp.dot(x9, w_ref[...], preferred_element_type=jnp.float32)
        ps, pq = _acc_stats(y, ps, pq)
        y_ref[j:j + sub] = y.reshape(sub, hw, cout).astype(jnp.bfloat16)
    ps_ref[0] = jnp.sum(ps, axis=0, keepdims=True)
    pq_ref[0] = jnp.sum(pq, axis=0, keepdims=True)


def _conv2_kernel(y1_ref, s_ref, t_ref, w_ref, y_ref, ps_ref, pq_ref,
                  *, h, w, sub):
    b, hw, c = y1_ref.shape
    cout = w_ref.shape[-1]
    ps = jnp.zeros((8, cout), jnp.float32)
    pq = jnp.zeros((8, cout), jnp.float32)
    for j in range(0, b, sub):
        v = y1_ref[j:j + sub].astype(jnp.float32)
        hr = jnp.maximum(v * s_ref[...] + t_ref[...], 0.0).astype(jnp.bfloat16)
        x9 = _taps(hr.reshape(sub, h, w, c), h, w)
        y = jnp.dot(x9, w_ref[...], preferred_element_type=jnp.float32)
        ps, pq = _acc_stats(y, ps, pq)
        y_ref[j:j + sub] = y.reshape(sub, hw, cout).astype(jnp.bfloat16)
    ps_ref[0] = jnp.sum(ps, axis=0, keepdims=True)
    pq_ref[0] = jnp.sum(pq, axis=0, keepdims=True)


def _epi_kernel(y2_ref, s_ref, t_ref, down_ref, pool_ref, rp_ref, *, h, w):
    b, hw, c = y2_ref.shape
    d = jnp.maximum(
        y2_ref[...].astype(jnp.float32) * s_ref[...] + t_ref[...], 0.0)
    down_ref[...] = jnp.swapaxes(d, 1, 2)
    half = d.reshape(b * (h // 2), 2 * w, c)
    rp = jnp.maximum(half[:, :w, :], half[:, w:, :])
    rp_ref[...] = rp.reshape(b * (hw // 2), c)
    npool = b * (hw // 4)
    pr = jnp.maximum(rp_ref[pl.ds(0, npool, 2), :],
                     rp_ref[pl.ds(1, npool, 2), :])
    pool_ref[...] = jnp.swapaxes(pr.reshape(b, hw // 4, c), 1, 2)


def _conv1(x3, w1f, h, w, blk, sub):
    n, c, hw = x3.shape
    cout = w1f.shape[-1]
    nb = n // blk
    return pl.pallas_call(
        functools.partial(_conv1_kernel, h=h, w=w, sub=sub),
        grid=(nb,),
        in_specs=[
            pl.BlockSpec((blk, c, hw), lambda i: (i, 0, 0)),
            pl.BlockSpec((9 * c, cout), lambda i: (0, 0)),
        ],
        out_specs=(
            pl.BlockSpec((blk, hw, cout), lambda i: (i, 0, 0)),
            pl.BlockSpec((1, 1, cout), lambda i: (i, 0, 0)),
            pl.BlockSpec((1, 1, cout), lambda i: (i, 0, 0)),
        ),
        out_shape=(
            jax.ShapeDtypeStruct((n, hw, cout), jnp.bfloat16),
            jax.ShapeDtypeStruct((nb, 1, cout), jnp.float32),
            jax.ShapeDtypeStruct((nb, 1, cout), jnp.float32),
        ),
        compiler_params=pltpu.CompilerParams(
            dimension_semantics=("parallel",)),
    )(x3, w1f)


def _conv2(y1, sc, sh, w2f, h, w, blk, sub):
    n, hw, c = y1.shape
    cout = w2f.shape[-1]
    nb = n // blk
    vec = pl.BlockSpec((1, c), lambda i: (0, 0))
    return pl.pallas_call(
        functools.partial(_conv2_kernel, h=h, w=w, sub=sub),
        grid=(nb,),
        in_specs=[
            pl.BlockSpec((blk, hw, c), lambda i: (i, 0, 0)),
            vec, vec,
            pl.BlockSpec((9 * c, cout), lambda i: (0, 0)),
        ],
        out_specs=(
            pl.BlockSpec((blk, hw, cout), lambda i: (i, 0, 0)),
            pl.BlockSpec((1, 1, cout), lambda i: (i, 0, 0)),
            pl.BlockSpec((1, 1, cout), lambda i: (i, 0, 0)),
        ),
        out_shape=(
            jax.ShapeDtypeStruct((n, hw, cout), jnp.bfloat16),
            jax.ShapeDtypeStruct((nb, 1, cout), jnp.float32),
            jax.ShapeDtypeStruct((nb, 1, cout), jnp.float32),
        ),
        compiler_params=pltpu.CompilerParams(
            dimension_semantics=("parallel",)),
    )(y1, sc, sh, w2f)


def _epi(y2, sc, sh, h, w, blk):
    n, hw, c = y2.shape
    nb = n // blk
    vec = pl.BlockSpec((1, c), lambda i: (0, 0))
    return pl.pallas_call(
        functools.partial(_epi_kernel, h=h, w=w),
        grid=(nb,),
        in_specs=[pl.BlockSpec((blk, hw, c), lambda i: (i, 0, 0)), vec, vec],
        out_specs=(
            pl.BlockSpec((blk, c, hw), lambda i: (i, 0, 0)),
            pl.BlockSpec((blk, c, hw // 4), lambda i: (i, 0, 0)),
        ),
        out_shape=(
            jax.ShapeDtypeStruct((n, c, hw), jnp.float32),
            jax.ShapeDtypeStruct((n, c, hw // 4), jnp.float32),
        ),
        scratch_shapes=[pltpu.VMEM((blk * hw // 2, c), jnp.float32)],
        compiler_params=pltpu.CompilerParams(
            dimension_semantics=("parallel",)),
    )(y2, sc, sh)


def _bn_affine(ps, pq, g, beta, m):
    s = jnp.sum(ps, axis=0)
    q = jnp.sum(pq, axis=0)
    mean = s / m
    var = q / m - mean * mean
    sc = g * jax.lax.rsqrt(var + _EPS)
    sh = beta - mean * sc
    return sc, sh


def kernel(x, w1, b1, g1, beta1, w2, b2, g2, beta2):
    n, cin, h, w = x.shape
    c1 = w1.shape[-1]
    c2 = w2.shape[-1]
    hw = h * w
    x3 = x.reshape(n, cin, hw)
    w1f = w1.reshape(9 * cin, c1).astype(jnp.bfloat16)
    w2f = w2.reshape(9 * c1, c2).astype(jnp.bfloat16)
    m = float(n * hw)

    blk = min(16, n)
    sub = min(4, blk)
    y1, ps1, pq1 = _conv1(x3, w1f, h, w, blk, sub)
    sc1, sh1 = _bn_affine(ps1, pq1, g1, beta1, m)
    y2, ps2, pq2 = _conv2(y1, sc1, sh1, w2f, h, w, blk, sub)
    sc2, sh2 = _bn_affine(ps2, pq2, g2, beta2, m)
    down, pool = _epi(y2, sc2, sh2, h, w, min(8, n))
    return (down.reshape(n, c2, h, w), pool.reshape(n, c2, h // 2, w // 2))
